# baseline (speedup 1.0000x reference)
import functools

import jax
import jax.numpy as jnp
import numpy as np
from jax import lax
from jax.experimental import pallas as pl
from jax.experimental.pallas import tpu as pltpu
from jax.experimental.pallas import tpu_sc as plsc


def _sc_flatten_idx(B, HIST):
    info = plsc.get_sparse_core_info()
    nc, ns = info.num_cores, info.num_subcores
    nw = nc * ns
    assert B % nw == 0
    bpw = B // nw
    n = bpw * HIST
    offs = sorted({min(16 * k, HIST - 16) for k in range((HIST + 15) // 16)})

    mesh = plsc.VectorSubcoreMesh(core_axis_name="c", subcore_axis_name="s")

    @functools.partial(
        pl.kernel,
        mesh=mesh,
        out_type=jax.ShapeDtypeStruct((B * HIST,), jnp.int32),
        scratch_types=[
            pltpu.VMEM((bpw, HIST), jnp.int32),
            pltpu.VMEM((n,), jnp.int32),
        ],
    )
    def sc_flat(idx_hbm, out_hbm, v2, flat):
        wid = lax.axis_index("s") * nc + lax.axis_index("c")
        pltpu.sync_copy(idx_hbm.at[pl.ds(wid * bpw, bpw)], v2)

        def body(j, carry):
            for o in offs:
                flat[pl.ds(j * HIST + o, 16)] = v2[j, pl.ds(o, 16)]
            return carry

        lax.fori_loop(0, bpw, body, 0)
        pltpu.sync_copy(flat, out_hbm.at[pl.ds(wid * n, n)])

    return sc_flat


def _sc_gather_sum(B, HIST, D, ROWW):
    info = plsc.get_sparse_core_info()
    nc, ns = info.num_cores, info.num_subcores
    nw = nc * ns
    assert B % nw == 0
    bpw = B // nw

    n_vec = D // 16

    G = 1
    NBUF = 4
    GH = G * HIST
    NG = bpw // G
    U = 8
    assert bpw % (G * NBUF) == 0 and HIST % U == 0 and GH % 8 == 0
    chunks = [(o, min(128, GH - o)) for o in range(0, GH, 128)]
    assert all(o % 8 == 0 for o, _ in chunks)

    mesh = plsc.VectorSubcoreMesh(core_axis_name="c", subcore_axis_name="s")

    @functools.partial(
        pl.kernel,
        mesh=mesh,
        compiler_params=pltpu.CompilerParams(use_tc_tiling_on_sc=False,
                                             needs_layout_passes=False),
        out_type=jax.ShapeDtypeStruct((B, D), jnp.float32),
        scratch_types=[
            pltpu.VMEM((bpw * HIST,), jnp.int32),
            pltpu.VMEM((NBUF, GH, ROWW), jnp.bfloat16),
            pltpu.VMEM((bpw, D), jnp.float32),
        ] + [pltpu.SemaphoreType.DMA] * NBUF,
    )
    def sc_sum(idx_hbm, table_hbm, out_hbm, idx_v, rows_v, stage_v, *sems):
        wid = lax.axis_index("s") * nc + lax.axis_index("c")
        base = wid * bpw
        pltpu.sync_copy(idx_hbm.at[pl.ds(base * HIST, bpw * HIST)], idx_v)

        def _copies(g, buf):
            off = pl.multiple_of(g * GH, 8)
            return [pltpu.make_async_copy(
                        table_hbm.at[idx_v.at[pl.ds(off + o, l)]],
                        rows_v.at[buf, pl.ds(o, l)],
                        sems[buf])
                    for o, l in chunks]

        def issue(g, buf):
            for c in _copies(g, buf):
                c.start()

        def drain(g, buf):
            for c in _copies(g, buf):
                c.wait()

        zero = jnp.zeros((16,), jnp.float32)

        def accum(g, buf):
            for rr in range(G):
                def body(jj, accs, _rr=rr):
                    j0 = _rr * HIST + jj * U
                    for u in range(U):
                        na = []
                        for h in range(n_vec // 2):
                            x = rows_v[buf, j0 + u, pl.ds(32 * h, 32)]
                            a, bv = plsc.unpack(
                                x, format=plsc.PackFormat.INTERLEAVED)
                            na.append(accs[2 * h] + a)
                            na.append(accs[2 * h + 1] + bv)
                        accs = tuple(na)
                    return accs

                accs = lax.fori_loop(0, HIST // U, body, (zero,) * n_vec)
                r_out = g * G + rr
                for k in range(n_vec):
                    stage_v[r_out, pl.ds(16 * k, 16)] = accs[k]

        for p in range(NBUF - 1):
            issue(p, p)

        def outer(i, carry):
            g0 = i * NBUF
            for b in range(NBUF):
                cur = g0 + b
                nxt = cur + NBUF - 1

                @pl.when(nxt < NG)
                def _(nxt=nxt, b=b):
                    issue(nxt, (b + NBUF - 1) % NBUF)

                drain(cur, b)
                accum(cur, b)
            return carry

        lax.fori_loop(0, NG // NBUF, outer, 0)
        pltpu.sync_copy(stage_v, out_hbm.at[pl.ds(base, bpw)])

    return sc_sum


def _tc_linear(sums, W, b2, scale):
    B, D = sums.shape
    OUT = W.shape[1]
    blk = 512 if B % 512 == 0 else B

    def body(s_ref, w_ref, b_ref, o_ref):
        o_ref[...] = jnp.dot(s_ref[...] * scale, w_ref[...],
                             preferred_element_type=jnp.float32) + b_ref[...]

    return pl.pallas_call(
        body,
        grid=(B // blk,),
        in_specs=[
            pl.BlockSpec((blk, D), lambda i: (i, 0)),
            pl.BlockSpec((D, OUT), lambda i: (0, 0)),
            pl.BlockSpec((1, OUT), lambda i: (0, 0)),
        ],
        out_specs=pl.BlockSpec((blk, OUT), lambda i: (i, 0)),
        out_shape=jax.ShapeDtypeStruct((B, OUT), jnp.float32),
    )(sums, W, b2)


def kernel(word_indices, embedding, W, b):
    B, HIST = word_indices.shape
    D = embedding.shape[1]
    idx_flat = _sc_flatten_idx(B, HIST)(word_indices.astype(jnp.int32))
    sums = _sc_gather_sum(B, HIST, D, D)(idx_flat,
                                         embedding.astype(jnp.bfloat16))
    perm = np.concatenate(
        [g * 32 + np.concatenate([np.arange(0, 32, 2), np.arange(1, 32, 2)])
         for g in range(D // 32)])
    return _tc_linear(sums, W[perm, :], b.reshape(1, -1), 1.0 / HIST)


# speedup vs baseline: 1.2991x; 1.2991x over previous
import functools

import jax
import jax.numpy as jnp
from jax import lax
from jax.experimental import pallas as pl
from jax.experimental.pallas import tpu as pltpu
from jax.experimental.pallas import tpu_sc as plsc


def _sc_flatten_idx(B, HIST):
    info = plsc.get_sparse_core_info()
    nc, ns = info.num_cores, info.num_subcores
    nw = nc * ns
    assert B % nw == 0
    bpw = B // nw
    n = bpw * HIST
    offs = sorted({min(16 * k, HIST - 16) for k in range((HIST + 15) // 16)})

    mesh = plsc.VectorSubcoreMesh(core_axis_name="c", subcore_axis_name="s")

    @functools.partial(
        pl.kernel,
        mesh=mesh,
        out_type=jax.ShapeDtypeStruct((B * HIST,), jnp.int32),
        scratch_types=[
            pltpu.VMEM((bpw, HIST), jnp.int32),
            pltpu.VMEM((n,), jnp.int32),
        ],
    )
    def sc_flat(idx_hbm, out_hbm, v2, flat):
        wid = lax.axis_index("s") * nc + lax.axis_index("c")
        pltpu.sync_copy(idx_hbm.at[pl.ds(wid * bpw, bpw)], v2)

        def body(j, carry):
            for o in offs:
                flat[pl.ds(j * HIST + o, 16)] = v2[j, pl.ds(o, 16)]
            return carry

        lax.fori_loop(0, bpw, body, 0)
        pltpu.sync_copy(flat, out_hbm.at[pl.ds(wid * n, n)])

    return sc_flat


def _sc_gather_sum(B, HIST, D, ROWW):
    info = plsc.get_sparse_core_info()
    nc, ns = info.num_cores, info.num_subcores
    nw = nc * ns
    assert B % nw == 0
    bpw = B // nw

    n_vec = D // 16

    G = 1
    NBUF = 4
    GH = G * HIST
    NG = bpw // G
    U = 8
    assert bpw % (G * NBUF) == 0 and HIST % U == 0 and GH % 8 == 0
    chunks = [(o, min(128, GH - o)) for o in range(0, GH, 128)]
    assert all(o % 8 == 0 for o, _ in chunks)

    mesh = plsc.VectorSubcoreMesh(core_axis_name="c", subcore_axis_name="s")

    @functools.partial(
        pl.kernel,
        mesh=mesh,
        compiler_params=pltpu.CompilerParams(use_tc_tiling_on_sc=False),
        out_type=jax.ShapeDtypeStruct((B, D), jnp.float32),
        scratch_types=[
            pltpu.VMEM((bpw * HIST,), jnp.int32),
            pltpu.VMEM((NBUF, GH, ROWW), jnp.float32),
            pltpu.VMEM((bpw, D), jnp.float32),
        ] + [pltpu.SemaphoreType.DMA] * NBUF,
    )
    def sc_sum(idx_hbm, table_hbm, out_hbm, idx_v, rows_v, stage_v, *sems):
        wid = lax.axis_index("s") * nc + lax.axis_index("c")
        base = wid * bpw
        pltpu.sync_copy(idx_hbm.at[pl.ds(base * HIST, bpw * HIST)], idx_v)

        def _copies(g, buf):
            off = pl.multiple_of(g * GH, 8)
            return [pltpu.make_async_copy(
                        table_hbm.at[idx_v.at[pl.ds(off + o, l)]],
                        rows_v.at[buf, pl.ds(o, l)],
                        sems[buf])
                    for o, l in chunks]

        def issue(g, buf):
            for c in _copies(g, buf):
                c.start()

        def drain(g, buf):
            for c in _copies(g, buf):
                c.wait()

        zero = jnp.zeros((16,), jnp.float32)

        def accum(g, buf):
            for rr in range(G):
                def body(jj, accs, _rr=rr):
                    j0 = _rr * HIST + jj * U
                    for u in range(U):
                        accs = tuple(
                            accs[k] + rows_v[buf, j0 + u, pl.ds(16 * k, 16)]
                            for k in range(n_vec))
                    return accs

                accs = lax.fori_loop(0, HIST // U, body, (zero,) * n_vec)
                r_out = g * G + rr
                for k in range(n_vec):
                    stage_v[r_out, pl.ds(16 * k, 16)] = accs[k]

        for p in range(NBUF - 1):
            issue(p, p)

        def outer(i, carry):
            g0 = i * NBUF
            for b in range(NBUF):
                cur = g0 + b
                nxt = cur + NBUF - 1

                @pl.when(nxt < NG)
                def _(nxt=nxt, b=b):
                    issue(nxt, (b + NBUF - 1) % NBUF)

                drain(cur, b)
                accum(cur, b)
            return carry

        lax.fori_loop(0, NG // NBUF, outer, 0)
        pltpu.sync_copy(stage_v, out_hbm.at[pl.ds(base, bpw)])

    return sc_sum


def _tc_linear(sums, W, b2, scale):
    B, D = sums.shape
    OUT = W.shape[1]
    blk = 512 if B % 512 == 0 else B

    def body(s_ref, w_ref, b_ref, o_ref):
        o_ref[...] = jnp.dot(s_ref[...] * scale, w_ref[...],
                             preferred_element_type=jnp.float32) + b_ref[...]

    return pl.pallas_call(
        body,
        grid=(B // blk,),
        in_specs=[
            pl.BlockSpec((blk, D), lambda i: (i, 0)),
            pl.BlockSpec((D, OUT), lambda i: (0, 0)),
            pl.BlockSpec((1, OUT), lambda i: (0, 0)),
        ],
        out_specs=pl.BlockSpec((blk, OUT), lambda i: (i, 0)),
        out_shape=jax.ShapeDtypeStruct((B, OUT), jnp.float32),
    )(sums, W, b2)


def kernel(word_indices, embedding, W, b):
    B, HIST = word_indices.shape
    D = embedding.shape[1]
    idx_flat = _sc_flatten_idx(B, HIST)(word_indices.astype(jnp.int32))
    sums = _sc_gather_sum(B, HIST, D, D)(idx_flat, embedding)
    return _tc_linear(sums, W, b.reshape(1, -1), 1.0 / HIST)
